# Initial kernel scaffold; baseline (speedup 1.0000x reference)
import jax
import jax.numpy as jnp
from jax.experimental import pallas as pl


def kernel(x, edge_index, W1, b1, W2, b2):
    raise NotImplementedError("write your pallas kernel here")


# speedup vs baseline: 8.2796x; 8.2796x over previous
import functools

import jax
import jax.numpy as jnp
from jax import lax
from jax.experimental import pallas as pl
from jax.experimental.pallas import tpu as pltpu
from jax.experimental.pallas import tpu_sc as plsc

N = 10000
F_IN = 256
HIDDEN = 64
F_OUT = 128
ALPHA = 0.1
K_PROP = 10

NC = 2
NS = 16
LANES = 16

N_PAD = 10240
ROWS_PT = N_PAD // NS
RB = 128
E_PAD = 172032
EBLK = 128
EB = E_PAD // (NS * EBLK)
EBW = E_PAD // (NC * NS * EBLK)
DUMMY = N


def _sc_mesh():
    return plsc.VectorSubcoreMesh(
        core_axis_name="c", subcore_axis_name="s", num_cores=NC, num_subcores=NS
    )


def _deg_body(dst_hbm, deg_out, dst_t, ones_t, zbuf, s_deg):
    c = lax.axis_index("c")
    s = lax.axis_index("s")
    w = s * NC + c
    r0 = s * ROWS_PT

    for f in range(RB // LANES):
        zbuf[pl.ds(LANES * f, LANES)] = jnp.zeros((LANES,), jnp.float32)
        ones_t[pl.ds(LANES * f, LANES)] = jnp.ones((LANES,), jnp.float32)
    for z in range(ROWS_PT // RB):
        pltpu.sync_copy(zbuf, s_deg.at[pl.ds(r0 + z * RB, RB)])
    pltpu.sync_copy(dst_hbm.at[w], dst_t)
    plsc.subcore_barrier()

    @pl.loop(0, EBW)
    def _edges(j):
        pltpu.sync_copy(ones_t, s_deg.at[dst_t.at[j]], add=True)

    plsc.subcore_barrier()
    pltpu.sync_copy(s_deg.at[pl.ds(r0, ROWS_PT)], deg_out.at[c, pl.ds(r0, ROWS_PT)])


_SC_PARAMS = pltpu.CompilerParams(use_tc_tiling_on_sc=False)

_deg_call = pl.kernel(
    _deg_body,
    out_type=jax.ShapeDtypeStruct((NC, N_PAD), jnp.float32),
    mesh=_sc_mesh(),
    compiler_params=_SC_PARAMS,
    scratch_types=[
        pltpu.VMEM((EBW, EBLK), jnp.int32),
        pltpu.VMEM((EBLK,), jnp.float32),
        pltpu.VMEM((RB,), jnp.float32),
        pltpu.VMEM_SHARED((N_PAD,), jnp.float32),
    ],
)


def _prop_body(Fc, u0_hbm, u0c_hbm, w2_hbm, src_hbm, dst_hbm, u_hbm,
               src_t, dst_t, gbuf, nbuf, zbuf, w2_t, u0c_t, s_s):
    c = lax.axis_index("c")
    s = lax.axis_index("s")
    r0 = s * ROWS_PT
    base = c * N_PAD + r0
    NF = Fc // LANES

    pltpu.sync_copy(src_hbm.at[s], src_t)
    pltpu.sync_copy(dst_hbm.at[s], dst_t)
    pltpu.sync_copy(w2_hbm.at[pl.ds(r0, ROWS_PT)], w2_t)
    pltpu.sync_copy(u0c_hbm.at[pl.ds(base, ROWS_PT)], u0c_t)

    off = (c * N_PAD).astype(jnp.int32)

    @pl.loop(0, EB)
    def _adj(j):
        for f in range(EBLK // LANES):
            sl = pl.ds(LANES * f, LANES)
            src_t[j, sl] = src_t[j, sl] + off

    @pl.loop(0, RB)
    def _zero(i):
        for f in range(NF):
            zbuf[i, pl.ds(LANES * f, LANES)] = jnp.zeros((LANES,), jnp.float32)

    for z in range(ROWS_PT // RB):
        pltpu.sync_copy(u0_hbm.at[pl.ds(base + z * RB, RB)], nbuf)
        pltpu.sync_copy(nbuf, u_hbm.at[pl.ds(base + z * RB, RB)])

    plsc.subcore_barrier()

    @pl.loop(0, K_PROP)
    def _step(_k):
        for z in range(ROWS_PT // RB):
            pltpu.sync_copy(zbuf, s_s.at[pl.ds(r0 + z * RB, RB)])
        plsc.subcore_barrier()

        @pl.loop(0, EB)
        def _edges(j):
            pltpu.sync_copy(u_hbm.at[src_t.at[j]], gbuf)
            pltpu.sync_copy(gbuf, s_s.at[dst_t.at[j]], add=True)

        plsc.subcore_barrier()

        @pl.loop(0, ROWS_PT // RB)
        def _node(z):
            pltpu.sync_copy(s_s.at[pl.ds(r0 + z * RB, RB)], nbuf)

            @pl.loop(0, RB // LANES)
            def _rowg(b):
                wvec = w2_t[pl.ds(z * RB + b * LANES, LANES)]
                for r in range(LANES):
                    i = b * LANES + r
                    wv = wvec[r]
                    for f in range(NF):
                        sl = pl.ds(LANES * f, LANES)
                        nbuf[i, sl] = nbuf[i, sl] * wv + u0c_t[z * RB + i, sl]

            pltpu.sync_copy(nbuf, u_hbm.at[pl.ds(base + z * RB, RB)])

        plsc.subcore_barrier()


def _make_prop(Fc):
    return pl.kernel(
        functools.partial(_prop_body, Fc),
        out_type=jax.ShapeDtypeStruct((NC * N_PAD, Fc), jnp.float32),
        mesh=_sc_mesh(),
        compiler_params=_SC_PARAMS,
        scratch_types=[
            pltpu.VMEM((EB, EBLK), jnp.int32),
            pltpu.VMEM((EB, EBLK), jnp.int32),
            pltpu.VMEM((EBLK, Fc), jnp.float32),
            pltpu.VMEM((RB, Fc), jnp.float32),
            pltpu.VMEM((RB, Fc), jnp.float32),
            pltpu.VMEM((ROWS_PT,), jnp.float32),
            pltpu.VMEM((ROWS_PT, Fc), jnp.float32),
            pltpu.VMEM_SHARED((N_PAD, Fc), jnp.float32),
        ],
    )


_prop32 = _make_prop(HIDDEN // NC)
_prop64 = _make_prop(F_OUT // NC)


_TCR = 512


def _tc1_body(degT_ref, x_ref, w1_ref, b1_ref,
              u0_ref, u0c_ref, w2_ref, sqd_ref, dis_ref):
    deg = degT_ref[:, 0:1] + degT_ref[:, 1:2]
    dis = jnp.where(deg > 0.0, lax.rsqrt(deg), 0.0)
    h0 = lax.dot_general(
        x_ref[...], w1_ref[...], (((1,), (0,)), ((), ())),
        preferred_element_type=jnp.float32) + b1_ref[...]
    u0 = h0 * dis
    u0_ref[...] = u0
    u0c_ref[...] = ALPHA * u0
    w2_ref[...] = (1.0 - ALPHA) * dis * dis
    sqd_ref[...] = jnp.sqrt(deg)
    dis_ref[...] = dis


def _tc1(degT, x_pad, W1, b1):
    grid = (N_PAD // _TCR,)
    return pl.pallas_call(
        _tc1_body,
        grid=grid,
        in_specs=[
            pl.BlockSpec((_TCR, 2), lambda i: (i, 0)),
            pl.BlockSpec((_TCR, F_IN), lambda i: (i, 0)),
            pl.BlockSpec((F_IN, HIDDEN), lambda i: (0, 0)),
            pl.BlockSpec((1, HIDDEN), lambda i: (0, 0)),
        ],
        out_specs=[
            pl.BlockSpec((_TCR, HIDDEN), lambda i: (i, 0)),
            pl.BlockSpec((_TCR, HIDDEN), lambda i: (i, 0)),
            pl.BlockSpec((_TCR, 1), lambda i: (i, 0)),
            pl.BlockSpec((_TCR, 1), lambda i: (i, 0)),
            pl.BlockSpec((_TCR, 1), lambda i: (i, 0)),
        ],
        out_shape=[
            jax.ShapeDtypeStruct((N_PAD, HIDDEN), jnp.float32),
            jax.ShapeDtypeStruct((N_PAD, HIDDEN), jnp.float32),
            jax.ShapeDtypeStruct((N_PAD, 1), jnp.float32),
            jax.ShapeDtypeStruct((N_PAD, 1), jnp.float32),
            jax.ShapeDtypeStruct((N_PAD, 1), jnp.float32),
        ],
    )(degT, x_pad, W1, b1)


def _tc2_body(u1_ref, sqd_ref, dis_ref, w2_ref, b2_ref, u02_ref, u0c2_ref):
    h1 = jnp.maximum(u1_ref[...] * sqd_ref[...], 0.0)
    h02 = lax.dot_general(
        h1, w2_ref[...], (((1,), (0,)), ((), ())),
        preferred_element_type=jnp.float32) + b2_ref[...]
    u02 = h02 * dis_ref[...]
    u02_ref[...] = u02
    u0c2_ref[...] = ALPHA * u02


def _tc2(u1, sqd, dis, W2, b2):
    grid = (N_PAD // _TCR,)
    return pl.pallas_call(
        _tc2_body,
        grid=grid,
        in_specs=[
            pl.BlockSpec((_TCR, HIDDEN), lambda i: (i, 0)),
            pl.BlockSpec((_TCR, 1), lambda i: (i, 0)),
            pl.BlockSpec((_TCR, 1), lambda i: (i, 0)),
            pl.BlockSpec((HIDDEN, F_OUT), lambda i: (0, 0)),
            pl.BlockSpec((1, F_OUT), lambda i: (0, 0)),
        ],
        out_specs=[
            pl.BlockSpec((_TCR, F_OUT), lambda i: (i, 0)),
            pl.BlockSpec((_TCR, F_OUT), lambda i: (i, 0)),
        ],
        out_shape=[
            jax.ShapeDtypeStruct((N_PAD, F_OUT), jnp.float32),
            jax.ShapeDtypeStruct((N_PAD, F_OUT), jnp.float32),
        ],
    )(u1, sqd, dis, W2, b2)


def _tc3_body(u2_ref, sqd_ref, out_ref):
    out_ref[...] = u2_ref[...] * sqd_ref[...]


def _tc3(u2, sqd):
    grid = (N_PAD // _TCR,)
    return pl.pallas_call(
        _tc3_body,
        grid=grid,
        in_specs=[
            pl.BlockSpec((_TCR, F_OUT), lambda i: (i, 0)),
            pl.BlockSpec((_TCR, 1), lambda i: (i, 0)),
        ],
        out_specs=pl.BlockSpec((_TCR, F_OUT), lambda i: (i, 0)),
        out_shape=jax.ShapeDtypeStruct((N_PAD, F_OUT), jnp.float32),
    )(u2, sqd)


def _split_cols(a, Fc):
    return a.reshape(N_PAD, NC, Fc).transpose(1, 0, 2).reshape(NC * N_PAD, Fc)


def _merge_cols(a, Fc):
    return a.reshape(NC, N_PAD, Fc).transpose(1, 0, 2).reshape(N_PAD, NC * Fc)


@jax.jit
def kernel(x, edge_index, W1, b1, W2, b2):
    ei = edge_index.astype(jnp.int32)
    loop = jnp.arange(N, dtype=jnp.int32)
    src = jnp.concatenate([ei[0], loop])
    dst = jnp.concatenate([ei[1], loop])
    pad = E_PAD - src.shape[0]
    src = jnp.concatenate([src, jnp.full((pad,), DUMMY, jnp.int32)])
    dst = jnp.concatenate([dst, jnp.full((pad,), DUMMY, jnp.int32)])
    src_p = src.reshape(NS, EB, EBLK)
    dst_p = dst.reshape(NS, EB, EBLK)
    dst_w = dst.reshape(NC * NS, EBW, EBLK)
    x_pad = jnp.pad(x, ((0, N_PAD - N), (0, 0)))

    deg2 = _deg_call(dst_w)
    u0, u0c, w2c, sqd, dis = _tc1(deg2.T, x_pad, W1, b1.reshape(1, HIDDEN))
    w2v = w2c.reshape(N_PAD)

    u1 = _prop32(_split_cols(u0, HIDDEN // NC), _split_cols(u0c, HIDDEN // NC),
                 w2v, src_p, dst_p)
    u02, u0c2 = _tc2(_merge_cols(u1, HIDDEN // NC), sqd, dis,
                     W2, b2.reshape(1, F_OUT))
    u2 = _prop64(_split_cols(u02, F_OUT // NC), _split_cols(u0c2, F_OUT // NC),
                 w2v, src_p, dst_p)
    out = _tc3(_merge_cols(u2, F_OUT // NC), sqd)
    return out[:N]


# speedup vs baseline: 9.7420x; 1.1766x over previous
import functools

import jax
import jax.numpy as jnp
from jax import lax
from jax.experimental import pallas as pl
from jax.experimental.pallas import tpu as pltpu
from jax.experimental.pallas import tpu_sc as plsc

N = 10000
F_IN = 256
HIDDEN = 64
F_OUT = 128
ALPHA = 0.1
K_PROP = 10

NC = 2
NS = 16
LANES = 16

N_PAD = 10240
ROWS_PT = N_PAD // NS
RB = 128
E_PAD = 172032
EBLK = 128
EB = E_PAD // (NS * EBLK)
EBW = E_PAD // (NC * NS * EBLK)
DUMMY = N


def _sc_mesh():
    return plsc.VectorSubcoreMesh(
        core_axis_name="c", subcore_axis_name="s", num_cores=NC, num_subcores=NS
    )


def _deg_body(dst_hbm, deg_out, dst_t, ones_t, zbuf, s_deg):
    c = lax.axis_index("c")
    s = lax.axis_index("s")
    w = s * NC + c
    r0 = s * ROWS_PT

    for f in range(RB // LANES):
        zbuf[pl.ds(LANES * f, LANES)] = jnp.zeros((LANES,), jnp.float32)
        ones_t[pl.ds(LANES * f, LANES)] = jnp.ones((LANES,), jnp.float32)
    for z in range(ROWS_PT // RB):
        pltpu.sync_copy(zbuf, s_deg.at[pl.ds(r0 + z * RB, RB)])
    pltpu.sync_copy(dst_hbm.at[w], dst_t)
    plsc.subcore_barrier()

    @pl.loop(0, EBW)
    def _edges(j):
        pltpu.sync_copy(ones_t, s_deg.at[dst_t.at[j]], add=True)

    plsc.subcore_barrier()
    pltpu.sync_copy(s_deg.at[pl.ds(r0, ROWS_PT)], deg_out.at[c, pl.ds(r0, ROWS_PT)])


_SC_PARAMS = pltpu.CompilerParams(use_tc_tiling_on_sc=False)

_deg_call = pl.kernel(
    _deg_body,
    out_type=jax.ShapeDtypeStruct((NC, N_PAD), jnp.float32),
    mesh=_sc_mesh(),
    compiler_params=_SC_PARAMS,
    scratch_types=[
        pltpu.VMEM((EBW, EBLK), jnp.int32),
        pltpu.VMEM((EBLK,), jnp.float32),
        pltpu.VMEM((RB,), jnp.float32),
        pltpu.VMEM_SHARED((N_PAD,), jnp.float32),
    ],
)


G = 2
NGRP = EB // G


def _prop_body(Fc, u0_hbm, u0c_hbm, w2_hbm, src_hbm, dst_hbm, u_hbm,
               src_t, dst_t, gbuf, nbuf, cbuf, zbuf, w2_t, s_s,
               gsemA, gsemB, ssemA, ssemB):
    c = lax.axis_index("c")
    s = lax.axis_index("s")
    r0 = s * ROWS_PT
    base = c * N_PAD + r0
    NF = Fc // LANES
    gsem = (gsemA, gsemB)
    ssem = (ssemA, ssemB)

    pltpu.sync_copy(src_hbm.at[s], src_t)
    pltpu.sync_copy(dst_hbm.at[s], dst_t)
    pltpu.sync_copy(w2_hbm.at[pl.ds(r0, ROWS_PT)], w2_t)

    off = (c * N_PAD).astype(jnp.int32)

    @pl.loop(0, EB)
    def _adj(j):
        for f in range(EBLK // LANES):
            sl = pl.ds(LANES * f, LANES)
            src_t[j, sl] = src_t[j, sl] + off

    @pl.loop(0, RB)
    def _zero(i):
        for f in range(NF):
            zbuf[i, pl.ds(LANES * f, LANES)] = jnp.zeros((LANES,), jnp.float32)

    for z in range(ROWS_PT // RB):
        pltpu.sync_copy(u0_hbm.at[pl.ds(base + z * RB, RB)], nbuf)
        pltpu.sync_copy(nbuf, u_hbm.at[pl.ds(base + z * RB, RB)])

    plsc.subcore_barrier()

    def fire_gathers(g, bank):
        for b in range(G):
            pltpu.async_copy(u_hbm.at[src_t.at[g * G + b]],
                             gbuf.at[bank * G + b], gsem[bank])

    def drain_gathers(g, bank):
        for b in range(G):
            pltpu.make_async_copy(u_hbm.at[src_t.at[g * G + b]],
                                  gbuf.at[bank * G + b], gsem[bank]).wait()

    def fire_scatters(g, bank):
        for b in range(G):
            pltpu.async_copy(gbuf.at[bank * G + b],
                             s_s.at[dst_t.at[g * G + b]], ssem[bank], add=True)

    def drain_scatters(g, bank):
        for b in range(G):
            pltpu.make_async_copy(gbuf.at[bank * G + b],
                                  s_s.at[dst_t.at[g * G + b]], ssem[bank]).wait()

    def process(g, bank, prefetch, drain_prev=True):
        ob = 1 - bank
        if prefetch:
            fire_gathers(g + 1, ob)
        drain_gathers(g, bank)
        fire_scatters(g, bank)
        drain_scatters(g, bank)

    @pl.loop(0, K_PROP)
    def _step(_k):
        for z in range(ROWS_PT // RB):
            pltpu.sync_copy(zbuf, s_s.at[pl.ds(r0 + z * RB, RB)])
        plsc.subcore_barrier()

        def efire(j, slot, sem):
            pltpu.async_copy(u_hbm.at[src_t.at[j]], gbuf.at[slot], sem)

        def ewait(j, slot, sem):
            pltpu.make_async_copy(u_hbm.at[src_t.at[j]], gbuf.at[slot], sem).wait()

        efire(0, 0, gsemA)

        @pl.loop(0, EB // 2 - 1)
        def _epair(jj):
            j0 = 2 * jj
            ewait(j0, 0, gsemA)
            efire(j0 + 1, 1, gsemB)
            pltpu.sync_copy(gbuf.at[0], s_s.at[dst_t.at[j0]], add=True)
            ewait(j0 + 1, 1, gsemB)
            efire(j0 + 2, 0, gsemA)
            pltpu.sync_copy(gbuf.at[1], s_s.at[dst_t.at[j0 + 1]], add=True)

        ewait(EB - 2, 0, gsemA)
        efire(EB - 1, 1, gsemB)
        pltpu.sync_copy(gbuf.at[0], s_s.at[dst_t.at[EB - 2]], add=True)
        ewait(EB - 1, 1, gsemB)
        pltpu.sync_copy(gbuf.at[1], s_s.at[dst_t.at[EB - 1]], add=True)

        plsc.subcore_barrier()

        @pl.loop(0, ROWS_PT // RB)
        def _node(z):
            pltpu.sync_copy(s_s.at[pl.ds(r0 + z * RB, RB)], nbuf)
            pltpu.sync_copy(u0c_hbm.at[pl.ds(base + z * RB, RB)], cbuf)

            @pl.loop(0, RB // LANES)
            def _rowg(b):
                wvec = w2_t[pl.ds(z * RB + b * LANES, LANES)]
                for r in range(LANES):
                    i = b * LANES + r
                    wv = wvec[r]
                    for f in range(NF):
                        sl = pl.ds(LANES * f, LANES)
                        nbuf[i, sl] = nbuf[i, sl] * wv + cbuf[i, sl]

            pltpu.sync_copy(nbuf, u_hbm.at[pl.ds(base + z * RB, RB)])

        plsc.subcore_barrier()


def _make_prop(Fc):
    return pl.kernel(
        functools.partial(_prop_body, Fc),
        out_type=jax.ShapeDtypeStruct((NC * N_PAD, Fc), jnp.float32),
        mesh=_sc_mesh(),
        compiler_params=_SC_PARAMS,
        scratch_types=[
            pltpu.VMEM((EB, EBLK), jnp.int32),
            pltpu.VMEM((EB, EBLK), jnp.int32),
            pltpu.VMEM((2 * G, EBLK, Fc), jnp.float32),
            pltpu.VMEM((RB, Fc), jnp.float32),
            pltpu.VMEM((RB, Fc), jnp.float32),
            pltpu.VMEM((RB, Fc), jnp.float32),
            pltpu.VMEM((ROWS_PT,), jnp.float32),
            pltpu.VMEM_SHARED((N_PAD, Fc), jnp.float32),
            pltpu.SemaphoreType.DMA,
            pltpu.SemaphoreType.DMA,
            pltpu.SemaphoreType.DMA,
            pltpu.SemaphoreType.DMA,
        ],
    )


_prop32 = _make_prop(HIDDEN // NC)
_prop64 = _make_prop(F_OUT // NC)


_TCR = 512


def _tc1_body(degT_ref, x_ref, w1_ref, b1_ref,
              u0_ref, u0c_ref, w2_ref, sqd_ref, dis_ref):
    deg = degT_ref[:, 0:1] + degT_ref[:, 1:2]
    dis = jnp.where(deg > 0.0, lax.rsqrt(deg), 0.0)
    h0 = lax.dot_general(
        x_ref[...], w1_ref[...], (((1,), (0,)), ((), ())),
        preferred_element_type=jnp.float32) + b1_ref[...]
    u0 = h0 * dis
    u0_ref[...] = u0
    u0c_ref[...] = ALPHA * u0
    w2_ref[...] = (1.0 - ALPHA) * dis * dis
    sqd_ref[...] = jnp.sqrt(deg)
    dis_ref[...] = dis


def _tc1(degT, x_pad, W1, b1):
    grid = (N_PAD // _TCR,)
    return pl.pallas_call(
        _tc1_body,
        grid=grid,
        in_specs=[
            pl.BlockSpec((_TCR, 2), lambda i: (i, 0)),
            pl.BlockSpec((_TCR, F_IN), lambda i: (i, 0)),
            pl.BlockSpec((F_IN, HIDDEN), lambda i: (0, 0)),
            pl.BlockSpec((1, HIDDEN), lambda i: (0, 0)),
        ],
        out_specs=[
            pl.BlockSpec((_TCR, HIDDEN), lambda i: (i, 0)),
            pl.BlockSpec((_TCR, HIDDEN), lambda i: (i, 0)),
            pl.BlockSpec((_TCR, 1), lambda i: (i, 0)),
            pl.BlockSpec((_TCR, 1), lambda i: (i, 0)),
            pl.BlockSpec((_TCR, 1), lambda i: (i, 0)),
        ],
        out_shape=[
            jax.ShapeDtypeStruct((N_PAD, HIDDEN), jnp.float32),
            jax.ShapeDtypeStruct((N_PAD, HIDDEN), jnp.float32),
            jax.ShapeDtypeStruct((N_PAD, 1), jnp.float32),
            jax.ShapeDtypeStruct((N_PAD, 1), jnp.float32),
            jax.ShapeDtypeStruct((N_PAD, 1), jnp.float32),
        ],
    )(degT, x_pad, W1, b1)


def _tc2_body(u1_ref, sqd_ref, dis_ref, w2_ref, b2_ref, u02_ref, u0c2_ref):
    h1 = jnp.maximum(u1_ref[...] * sqd_ref[...], 0.0)
    h02 = lax.dot_general(
        h1, w2_ref[...], (((1,), (0,)), ((), ())),
        preferred_element_type=jnp.float32) + b2_ref[...]
    u02 = h02 * dis_ref[...]
    u02_ref[...] = u02
    u0c2_ref[...] = ALPHA * u02


def _tc2(u1, sqd, dis, W2, b2):
    grid = (N_PAD // _TCR,)
    return pl.pallas_call(
        _tc2_body,
        grid=grid,
        in_specs=[
            pl.BlockSpec((_TCR, HIDDEN), lambda i: (i, 0)),
            pl.BlockSpec((_TCR, 1), lambda i: (i, 0)),
            pl.BlockSpec((_TCR, 1), lambda i: (i, 0)),
            pl.BlockSpec((HIDDEN, F_OUT), lambda i: (0, 0)),
            pl.BlockSpec((1, F_OUT), lambda i: (0, 0)),
        ],
        out_specs=[
            pl.BlockSpec((_TCR, F_OUT), lambda i: (i, 0)),
            pl.BlockSpec((_TCR, F_OUT), lambda i: (i, 0)),
        ],
        out_shape=[
            jax.ShapeDtypeStruct((N_PAD, F_OUT), jnp.float32),
            jax.ShapeDtypeStruct((N_PAD, F_OUT), jnp.float32),
        ],
    )(u1, sqd, dis, W2, b2)


def _tc3_body(u2_ref, sqd_ref, out_ref):
    out_ref[...] = u2_ref[...] * sqd_ref[...]


def _tc3(u2, sqd):
    grid = (N_PAD // _TCR,)
    return pl.pallas_call(
        _tc3_body,
        grid=grid,
        in_specs=[
            pl.BlockSpec((_TCR, F_OUT), lambda i: (i, 0)),
            pl.BlockSpec((_TCR, 1), lambda i: (i, 0)),
        ],
        out_specs=pl.BlockSpec((_TCR, F_OUT), lambda i: (i, 0)),
        out_shape=jax.ShapeDtypeStruct((N_PAD, F_OUT), jnp.float32),
    )(u2, sqd)


def _split_cols(a, Fc):
    return a.reshape(N_PAD, NC, Fc).transpose(1, 0, 2).reshape(NC * N_PAD, Fc)


def _merge_cols(a, Fc):
    return a.reshape(NC, N_PAD, Fc).transpose(1, 0, 2).reshape(N_PAD, NC * Fc)


@jax.jit
def kernel(x, edge_index, W1, b1, W2, b2):
    ei = edge_index.astype(jnp.int32)
    loop = jnp.arange(N, dtype=jnp.int32)
    src = jnp.concatenate([ei[0], loop])
    dst = jnp.concatenate([ei[1], loop])
    pad = E_PAD - src.shape[0]
    src = jnp.concatenate([src, jnp.full((pad,), DUMMY, jnp.int32)])
    dst = jnp.concatenate([dst, jnp.full((pad,), DUMMY, jnp.int32)])
    src_p = src.reshape(NS, EB, EBLK)
    dst_p = dst.reshape(NS, EB, EBLK)
    dst_w = dst.reshape(NC * NS, EBW, EBLK)
    x_pad = jnp.pad(x, ((0, N_PAD - N), (0, 0)))

    deg2 = _deg_call(dst_w)
    u0, u0c, w2c, sqd, dis = _tc1(deg2.T, x_pad, W1, b1.reshape(1, HIDDEN))
    w2v = w2c.reshape(N_PAD)

    u1 = _prop32(_split_cols(u0, HIDDEN // NC), _split_cols(u0c, HIDDEN // NC),
                 w2v, src_p, dst_p)
    u02, u0c2 = _tc2(_merge_cols(u1, HIDDEN // NC), sqd, dis,
                     W2, b2.reshape(1, F_OUT))
    u2 = _prop64(_split_cols(u02, F_OUT // NC), _split_cols(u0c2, F_OUT // NC),
                 w2v, src_p, dst_p)
    out = _tc3(_merge_cols(u2, F_OUT // NC), sqd)
    return out[:N]


# speedup vs baseline: 11.4908x; 1.1795x over previous
import functools

import jax
import jax.numpy as jnp
from jax import lax
from jax.experimental import pallas as pl
from jax.experimental.pallas import tpu as pltpu
from jax.experimental.pallas import tpu_sc as plsc

N = 10000
F_IN = 256
HIDDEN = 64
F_OUT = 128
ALPHA = 0.1
K_PROP = 10

NC = 2
NS = 16
LANES = 16

N_PAD = 10240
ROWS_PT = N_PAD // NS
RB = 128
E_PAD = 172032
EBLK = 128
EB = E_PAD // (NS * EBLK)
EBW = E_PAD // (NC * NS * EBLK)
DUMMY = N


def _sc_mesh():
    return plsc.VectorSubcoreMesh(
        core_axis_name="c", subcore_axis_name="s", num_cores=NC, num_subcores=NS
    )


def _deg_body(dst_hbm, deg_out, dst_t, ones_t, zbuf, s_deg):
    c = lax.axis_index("c")
    s = lax.axis_index("s")
    w = s * NC + c
    r0 = s * ROWS_PT

    for f in range(RB // LANES):
        zbuf[pl.ds(LANES * f, LANES)] = jnp.zeros((LANES,), jnp.float32)
        ones_t[pl.ds(LANES * f, LANES)] = jnp.ones((LANES,), jnp.float32)
    for z in range(ROWS_PT // RB):
        pltpu.sync_copy(zbuf, s_deg.at[pl.ds(r0 + z * RB, RB)])
    pltpu.sync_copy(dst_hbm.at[w], dst_t)
    plsc.subcore_barrier()

    @pl.loop(0, EBW)
    def _edges(j):
        pltpu.sync_copy(ones_t, s_deg.at[dst_t.at[j]], add=True)

    plsc.subcore_barrier()
    pltpu.sync_copy(s_deg.at[pl.ds(r0, ROWS_PT)], deg_out.at[c, pl.ds(r0, ROWS_PT)])


_SC_PARAMS = pltpu.CompilerParams(use_tc_tiling_on_sc=False)

_deg_call = pl.kernel(
    _deg_body,
    out_type=jax.ShapeDtypeStruct((NC, N_PAD), jnp.float32),
    mesh=_sc_mesh(),
    compiler_params=_SC_PARAMS,
    scratch_types=[
        pltpu.VMEM((EBW, EBLK), jnp.int32),
        pltpu.VMEM((EBLK,), jnp.float32),
        pltpu.VMEM((RB,), jnp.float32),
        pltpu.VMEM_SHARED((N_PAD,), jnp.float32),
    ],
)


EBG = 256
EBg = E_PAD // (NS * EBG)


def _prop_body(Fc, u0_hbm, u0c_hbm, w2_hbm, src_hbm, dst_hbm, u_hbm,
               src_t, dst_t, gbuf, nbuf, cbuf, zbuf, w2_t, s_s,
               gsemA, gsemB):
    c = lax.axis_index("c")
    s = lax.axis_index("s")
    r0 = s * ROWS_PT
    base = c * N_PAD + r0
    NF = Fc // LANES

    pltpu.sync_copy(src_hbm.at[s], src_t)
    pltpu.sync_copy(dst_hbm.at[s], dst_t)
    pltpu.sync_copy(w2_hbm.at[pl.ds(r0, ROWS_PT)], w2_t)

    off = (c * N_PAD).astype(jnp.int32)

    @pl.loop(0, EBg)
    def _adj(j):
        for f in range(EBG // LANES):
            sl = pl.ds(LANES * f, LANES)
            src_t[j, sl] = src_t[j, sl] + off

    @pl.loop(0, RB)
    def _zero(i):
        for f in range(NF):
            zbuf[i, pl.ds(LANES * f, LANES)] = jnp.zeros((LANES,), jnp.float32)

    for z in range(ROWS_PT // RB):
        pltpu.sync_copy(u0_hbm.at[pl.ds(base + z * RB, RB)], nbuf)
        pltpu.sync_copy(nbuf, u_hbm.at[pl.ds(base + z * RB, RB)])

    plsc.subcore_barrier()

    @pl.loop(0, K_PROP)
    def _step(_k):
        for z in range(ROWS_PT // RB):
            pltpu.sync_copy(zbuf, s_s.at[pl.ds(r0 + z * RB, RB)])
        plsc.subcore_barrier()

        def efire(j, slot, sem):
            pltpu.async_copy(u_hbm.at[src_t.at[j]], gbuf.at[slot], sem)

        def ewait(j, slot, sem):
            pltpu.make_async_copy(u_hbm.at[src_t.at[j]], gbuf.at[slot], sem).wait()

        def scat(j, slot, half):
            pltpu.sync_copy(gbuf.at[slot, pl.ds(half * EBLK, EBLK)],
                            s_s.at[dst_t.at[2 * j + half]], add=True)

        efire(0, 0, gsemA)

        @pl.loop(0, EBg // 2 - 1)
        def _epair(jj):
            j0 = 2 * jj
            ewait(j0, 0, gsemA)
            efire(j0 + 1, 1, gsemB)
            scat(j0, 0, 0)
            scat(j0, 0, 1)
            ewait(j0 + 1, 1, gsemB)
            efire(j0 + 2, 0, gsemA)
            scat(j0 + 1, 1, 0)
            scat(j0 + 1, 1, 1)

        ewait(EBg - 2, 0, gsemA)
        efire(EBg - 1, 1, gsemB)
        scat(EBg - 2, 0, 0)
        scat(EBg - 2, 0, 1)
        ewait(EBg - 1, 1, gsemB)
        scat(EBg - 1, 1, 0)
        scat(EBg - 1, 1, 1)

        plsc.subcore_barrier()

        @pl.loop(0, ROWS_PT // RB)
        def _node(z):
            pltpu.sync_copy(s_s.at[pl.ds(r0 + z * RB, RB)], nbuf)
            pltpu.sync_copy(u0c_hbm.at[pl.ds(base + z * RB, RB)], cbuf)

            @pl.loop(0, RB // LANES)
            def _rowg(b):
                wvec = w2_t[pl.ds(z * RB + b * LANES, LANES)]
                for r in range(LANES):
                    i = b * LANES + r
                    wv = wvec[r]
                    for f in range(NF):
                        sl = pl.ds(LANES * f, LANES)
                        nbuf[i, sl] = nbuf[i, sl] * wv + cbuf[i, sl]

            pltpu.sync_copy(nbuf, u_hbm.at[pl.ds(base + z * RB, RB)])

        plsc.subcore_barrier()


def _make_prop(Fc):
    return pl.kernel(
        functools.partial(_prop_body, Fc),
        out_type=jax.ShapeDtypeStruct((NC * N_PAD, Fc), jnp.float32),
        mesh=_sc_mesh(),
        compiler_params=_SC_PARAMS,
        scratch_types=[
            pltpu.VMEM((EBg, EBG), jnp.int32),
            pltpu.VMEM((EB, EBLK), jnp.int32),
            pltpu.VMEM((2, EBG, Fc), jnp.float32),
            pltpu.VMEM((RB, Fc), jnp.float32),
            pltpu.VMEM((RB, Fc), jnp.float32),
            pltpu.VMEM((RB, Fc), jnp.float32),
            pltpu.VMEM((ROWS_PT,), jnp.float32),
            pltpu.VMEM_SHARED((N_PAD, Fc), jnp.float32),
            pltpu.SemaphoreType.DMA,
            pltpu.SemaphoreType.DMA,
        ],
    )


_prop32 = _make_prop(HIDDEN // NC)
_prop64 = _make_prop(F_OUT // NC)


_TCR = 512


def _tc1_body(degT_ref, x_ref, w1_ref, b1_ref,
              u0_ref, u0c_ref, w2_ref, sqd_ref, dis_ref):
    deg = degT_ref[:, 0:1] + degT_ref[:, 1:2]
    dis = jnp.where(deg > 0.0, lax.rsqrt(deg), 0.0)
    h0 = lax.dot_general(
        x_ref[...], w1_ref[...], (((1,), (0,)), ((), ())),
        preferred_element_type=jnp.float32) + b1_ref[...]
    u0 = h0 * dis
    u0_ref[...] = u0
    u0c_ref[...] = ALPHA * u0
    w2_ref[...] = (1.0 - ALPHA) * dis * dis
    sqd_ref[...] = jnp.sqrt(deg)
    dis_ref[...] = dis


def _tc1(degT, x_pad, W1, b1):
    grid = (N_PAD // _TCR,)
    return pl.pallas_call(
        _tc1_body,
        grid=grid,
        in_specs=[
            pl.BlockSpec((_TCR, 2), lambda i: (i, 0)),
            pl.BlockSpec((_TCR, F_IN), lambda i: (i, 0)),
            pl.BlockSpec((F_IN, HIDDEN), lambda i: (0, 0)),
            pl.BlockSpec((1, HIDDEN), lambda i: (0, 0)),
        ],
        out_specs=[
            pl.BlockSpec((_TCR, HIDDEN), lambda i: (i, 0)),
            pl.BlockSpec((_TCR, HIDDEN), lambda i: (i, 0)),
            pl.BlockSpec((_TCR, 1), lambda i: (i, 0)),
            pl.BlockSpec((_TCR, 1), lambda i: (i, 0)),
            pl.BlockSpec((_TCR, 1), lambda i: (i, 0)),
        ],
        out_shape=[
            jax.ShapeDtypeStruct((N_PAD, HIDDEN), jnp.float32),
            jax.ShapeDtypeStruct((N_PAD, HIDDEN), jnp.float32),
            jax.ShapeDtypeStruct((N_PAD, 1), jnp.float32),
            jax.ShapeDtypeStruct((N_PAD, 1), jnp.float32),
            jax.ShapeDtypeStruct((N_PAD, 1), jnp.float32),
        ],
    )(degT, x_pad, W1, b1)


def _tc2_body(u1_ref, sqd_ref, dis_ref, w2_ref, b2_ref, u02_ref, u0c2_ref):
    h1 = jnp.maximum(u1_ref[...] * sqd_ref[...], 0.0)
    h02 = lax.dot_general(
        h1, w2_ref[...], (((1,), (0,)), ((), ())),
        preferred_element_type=jnp.float32) + b2_ref[...]
    u02 = h02 * dis_ref[...]
    u02_ref[...] = u02
    u0c2_ref[...] = ALPHA * u02


def _tc2(u1, sqd, dis, W2, b2):
    grid = (N_PAD // _TCR,)
    return pl.pallas_call(
        _tc2_body,
        grid=grid,
        in_specs=[
            pl.BlockSpec((_TCR, HIDDEN), lambda i: (i, 0)),
            pl.BlockSpec((_TCR, 1), lambda i: (i, 0)),
            pl.BlockSpec((_TCR, 1), lambda i: (i, 0)),
            pl.BlockSpec((HIDDEN, F_OUT), lambda i: (0, 0)),
            pl.BlockSpec((1, F_OUT), lambda i: (0, 0)),
        ],
        out_specs=[
            pl.BlockSpec((_TCR, F_OUT), lambda i: (i, 0)),
            pl.BlockSpec((_TCR, F_OUT), lambda i: (i, 0)),
        ],
        out_shape=[
            jax.ShapeDtypeStruct((N_PAD, F_OUT), jnp.float32),
            jax.ShapeDtypeStruct((N_PAD, F_OUT), jnp.float32),
        ],
    )(u1, sqd, dis, W2, b2)


def _tc3_body(u2_ref, sqd_ref, out_ref):
    out_ref[...] = u2_ref[...] * sqd_ref[...]


def _tc3(u2, sqd):
    grid = (N_PAD // _TCR,)
    return pl.pallas_call(
        _tc3_body,
        grid=grid,
        in_specs=[
            pl.BlockSpec((_TCR, F_OUT), lambda i: (i, 0)),
            pl.BlockSpec((_TCR, 1), lambda i: (i, 0)),
        ],
        out_specs=pl.BlockSpec((_TCR, F_OUT), lambda i: (i, 0)),
        out_shape=jax.ShapeDtypeStruct((N_PAD, F_OUT), jnp.float32),
    )(u2, sqd)


def _split_cols(a, Fc):
    return a.reshape(N_PAD, NC, Fc).transpose(1, 0, 2).reshape(NC * N_PAD, Fc)


def _merge_cols(a, Fc):
    return a.reshape(NC, N_PAD, Fc).transpose(1, 0, 2).reshape(N_PAD, NC * Fc)


@jax.jit
def kernel(x, edge_index, W1, b1, W2, b2):
    ei = edge_index.astype(jnp.int32)
    loop = jnp.arange(N, dtype=jnp.int32)
    src = jnp.concatenate([ei[0], loop])
    dst = jnp.concatenate([ei[1], loop])
    pad = E_PAD - src.shape[0]
    src = jnp.concatenate([src, jnp.full((pad,), DUMMY, jnp.int32)])
    dst = jnp.concatenate([dst, jnp.full((pad,), DUMMY, jnp.int32)])
    src_p = src.reshape(NS, EBg, EBG)
    dst_p = dst.reshape(NS, EB, EBLK)
    dst_w = dst.reshape(NC * NS, EBW, EBLK)
    x_pad = jnp.pad(x, ((0, N_PAD - N), (0, 0)))

    deg2 = _deg_call(dst_w)
    u0, u0c, w2c, sqd, dis = _tc1(deg2.T, x_pad, W1, b1.reshape(1, HIDDEN))
    w2v = w2c.reshape(N_PAD)

    u1 = _prop32(_split_cols(u0, HIDDEN // NC), _split_cols(u0c, HIDDEN // NC),
                 w2v, src_p, dst_p)
    u02, u0c2 = _tc2(_merge_cols(u1, HIDDEN // NC), sqd, dis,
                     W2, b2.reshape(1, F_OUT))
    u2 = _prop64(_split_cols(u02, F_OUT // NC), _split_cols(u0c2, F_OUT // NC),
                 w2v, src_p, dst_p)
    out = _tc3(_merge_cols(u2, F_OUT // NC), sqd)
    return out[:N]


# speedup vs baseline: 11.5051x; 1.0012x over previous
import functools

import jax
import jax.numpy as jnp
from jax import lax
from jax.experimental import pallas as pl
from jax.experimental.pallas import tpu as pltpu
from jax.experimental.pallas import tpu_sc as plsc

N = 10000
F_IN = 256
HIDDEN = 64
F_OUT = 128
ALPHA = 0.1
K_PROP = 10

NC = 2
NS = 16
LANES = 16

N_PAD = 10240
ROWS_PT = N_PAD // NS
RB = 128
E_PAD = 172032
EBLK = 128
EB = E_PAD // (NS * EBLK)
EBW = E_PAD // (NC * NS * EBLK)
DUMMY = N


def _sc_mesh():
    return plsc.VectorSubcoreMesh(
        core_axis_name="c", subcore_axis_name="s", num_cores=NC, num_subcores=NS
    )


def _deg_body(dst_hbm, deg_out, dst_t, ones_t, zbuf, s_deg):
    c = lax.axis_index("c")
    s = lax.axis_index("s")
    w = s * NC + c
    r0 = s * ROWS_PT

    for f in range(RB // LANES):
        zbuf[pl.ds(LANES * f, LANES)] = jnp.zeros((LANES,), jnp.float32)
        ones_t[pl.ds(LANES * f, LANES)] = jnp.ones((LANES,), jnp.float32)
    for z in range(ROWS_PT // RB):
        pltpu.sync_copy(zbuf, s_deg.at[pl.ds(r0 + z * RB, RB)])
    pltpu.sync_copy(dst_hbm.at[w], dst_t)
    plsc.subcore_barrier()

    @pl.loop(0, EBW)
    def _edges(j):
        pltpu.sync_copy(ones_t, s_deg.at[dst_t.at[j]], add=True)

    plsc.subcore_barrier()
    pltpu.sync_copy(s_deg.at[pl.ds(r0, ROWS_PT)], deg_out.at[c, pl.ds(r0, ROWS_PT)])


_SC_PARAMS = pltpu.CompilerParams(use_tc_tiling_on_sc=False)

_deg_call = pl.kernel(
    _deg_body,
    out_type=jax.ShapeDtypeStruct((NC, N_PAD), jnp.float32),
    mesh=_sc_mesh(),
    compiler_params=_SC_PARAMS,
    scratch_types=[
        pltpu.VMEM((EBW, EBLK), jnp.int32),
        pltpu.VMEM((EBLK,), jnp.float32),
        pltpu.VMEM((RB,), jnp.float32),
        pltpu.VMEM_SHARED((N_PAD,), jnp.float32),
    ],
)


EBG = 256
EBg = E_PAD // (NS * EBG)


def _prop_body(Fc, u0_hbm, u0c_hbm, w2_hbm, src_hbm, dst_hbm, u_hbm,
               src_t, dst_t, gbuf, nbuf, cbuf, zbuf, w2_t, s_s,
               gsemA, gsemB):
    c = lax.axis_index("c")
    s = lax.axis_index("s")
    r0 = s * ROWS_PT
    base = c * N_PAD + r0
    NF = Fc // LANES

    pltpu.sync_copy(src_hbm.at[s], src_t)
    pltpu.sync_copy(dst_hbm.at[s], dst_t)
    pltpu.sync_copy(w2_hbm.at[pl.ds(r0, ROWS_PT)], w2_t)

    off = (c * N_PAD).astype(jnp.int32)

    @pl.loop(0, EBg)
    def _adj(j):
        for f in range(EBG // LANES):
            sl = pl.ds(LANES * f, LANES)
            src_t[j, sl] = src_t[j, sl] + off

    @pl.loop(0, RB)
    def _zero(i):
        for f in range(NF):
            zbuf[i, pl.ds(LANES * f, LANES)] = jnp.zeros((LANES,), jnp.float32)

    for z in range(ROWS_PT // RB):
        pltpu.sync_copy(u0_hbm.at[pl.ds(base + z * RB, RB)], nbuf)
        pltpu.sync_copy(nbuf, u_hbm.at[pl.ds(base + z * RB, RB)])

    plsc.subcore_barrier()

    @pl.loop(0, K_PROP)
    def _step(_k):
        for z in range(ROWS_PT // RB):
            pltpu.sync_copy(zbuf, s_s.at[pl.ds(r0 + z * RB, RB)])
        plsc.subcore_barrier()

        def efire(j, slot, sem):
            pltpu.async_copy(u_hbm.at[src_t.at[j]], gbuf.at[slot], sem)

        def ewait(j, slot, sem):
            pltpu.make_async_copy(u_hbm.at[src_t.at[j]], gbuf.at[slot], sem).wait()

        def scat(j, slot, half):
            if half == 1:
                return
            pltpu.sync_copy(gbuf.at[slot], s_s.at[dst_t.at[j]], add=True)

        efire(0, 0, gsemA)

        @pl.loop(0, EBg // 2 - 1)
        def _epair(jj):
            j0 = 2 * jj
            ewait(j0, 0, gsemA)
            efire(j0 + 1, 1, gsemB)
            scat(j0, 0, 0)
            scat(j0, 0, 1)
            ewait(j0 + 1, 1, gsemB)
            efire(j0 + 2, 0, gsemA)
            scat(j0 + 1, 1, 0)
            scat(j0 + 1, 1, 1)

        ewait(EBg - 2, 0, gsemA)
        efire(EBg - 1, 1, gsemB)
        scat(EBg - 2, 0, 0)
        scat(EBg - 2, 0, 1)
        ewait(EBg - 1, 1, gsemB)
        scat(EBg - 1, 1, 0)
        scat(EBg - 1, 1, 1)

        plsc.subcore_barrier()

        @pl.loop(0, ROWS_PT // RB)
        def _node(z):
            pltpu.sync_copy(s_s.at[pl.ds(r0 + z * RB, RB)], nbuf)
            pltpu.sync_copy(u0c_hbm.at[pl.ds(base + z * RB, RB)], cbuf)

            @pl.loop(0, RB // LANES)
            def _rowg(b):
                wvec = w2_t[pl.ds(z * RB + b * LANES, LANES)]
                for r in range(LANES):
                    i = b * LANES + r
                    wv = wvec[r]
                    for f in range(NF):
                        sl = pl.ds(LANES * f, LANES)
                        nbuf[i, sl] = nbuf[i, sl] * wv + cbuf[i, sl]

            pltpu.sync_copy(nbuf, u_hbm.at[pl.ds(base + z * RB, RB)])

        plsc.subcore_barrier()


def _make_prop(Fc):
    return pl.kernel(
        functools.partial(_prop_body, Fc),
        out_type=jax.ShapeDtypeStruct((NC * N_PAD, Fc), jnp.float32),
        mesh=_sc_mesh(),
        compiler_params=_SC_PARAMS,
        scratch_types=[
            pltpu.VMEM((EBg, EBG), jnp.int32),
            pltpu.VMEM((EBg, EBG), jnp.int32),
            pltpu.VMEM((2, EBG, Fc), jnp.float32),
            pltpu.VMEM((RB, Fc), jnp.float32),
            pltpu.VMEM((RB, Fc), jnp.float32),
            pltpu.VMEM((RB, Fc), jnp.float32),
            pltpu.VMEM((ROWS_PT,), jnp.float32),
            pltpu.VMEM_SHARED((N_PAD, Fc), jnp.float32),
            pltpu.SemaphoreType.DMA,
            pltpu.SemaphoreType.DMA,
        ],
    )


_prop32 = _make_prop(HIDDEN // NC)
_prop64 = _make_prop(F_OUT // NC)


_TCR = 512


def _tc1_body(degT_ref, x_ref, w1_ref, b1_ref,
              u0_ref, u0c_ref, w2_ref, sqd_ref, dis_ref):
    deg = degT_ref[:, 0:1] + degT_ref[:, 1:2]
    dis = jnp.where(deg > 0.0, lax.rsqrt(deg), 0.0)
    h0 = lax.dot_general(
        x_ref[...], w1_ref[...], (((1,), (0,)), ((), ())),
        preferred_element_type=jnp.float32) + b1_ref[...]
    u0 = h0 * dis
    u0_ref[...] = u0
    u0c_ref[...] = ALPHA * u0
    w2_ref[...] = (1.0 - ALPHA) * dis * dis
    sqd_ref[...] = jnp.sqrt(deg)
    dis_ref[...] = dis


def _tc1(degT, x_pad, W1, b1):
    grid = (N_PAD // _TCR,)
    return pl.pallas_call(
        _tc1_body,
        grid=grid,
        in_specs=[
            pl.BlockSpec((_TCR, 2), lambda i: (i, 0)),
            pl.BlockSpec((_TCR, F_IN), lambda i: (i, 0)),
            pl.BlockSpec((F_IN, HIDDEN), lambda i: (0, 0)),
            pl.BlockSpec((1, HIDDEN), lambda i: (0, 0)),
        ],
        out_specs=[
            pl.BlockSpec((_TCR, HIDDEN), lambda i: (i, 0)),
            pl.BlockSpec((_TCR, HIDDEN), lambda i: (i, 0)),
            pl.BlockSpec((_TCR, 1), lambda i: (i, 0)),
            pl.BlockSpec((_TCR, 1), lambda i: (i, 0)),
            pl.BlockSpec((_TCR, 1), lambda i: (i, 0)),
        ],
        out_shape=[
            jax.ShapeDtypeStruct((N_PAD, HIDDEN), jnp.float32),
            jax.ShapeDtypeStruct((N_PAD, HIDDEN), jnp.float32),
            jax.ShapeDtypeStruct((N_PAD, 1), jnp.float32),
            jax.ShapeDtypeStruct((N_PAD, 1), jnp.float32),
            jax.ShapeDtypeStruct((N_PAD, 1), jnp.float32),
        ],
    )(degT, x_pad, W1, b1)


def _tc2_body(u1_ref, sqd_ref, dis_ref, w2_ref, b2_ref, u02_ref, u0c2_ref):
    h1 = jnp.maximum(u1_ref[...] * sqd_ref[...], 0.0)
    h02 = lax.dot_general(
        h1, w2_ref[...], (((1,), (0,)), ((), ())),
        preferred_element_type=jnp.float32) + b2_ref[...]
    u02 = h02 * dis_ref[...]
    u02_ref[...] = u02
    u0c2_ref[...] = ALPHA * u02


def _tc2(u1, sqd, dis, W2, b2):
    grid = (N_PAD // _TCR,)
    return pl.pallas_call(
        _tc2_body,
        grid=grid,
        in_specs=[
            pl.BlockSpec((_TCR, HIDDEN), lambda i: (i, 0)),
            pl.BlockSpec((_TCR, 1), lambda i: (i, 0)),
            pl.BlockSpec((_TCR, 1), lambda i: (i, 0)),
            pl.BlockSpec((HIDDEN, F_OUT), lambda i: (0, 0)),
            pl.BlockSpec((1, F_OUT), lambda i: (0, 0)),
        ],
        out_specs=[
            pl.BlockSpec((_TCR, F_OUT), lambda i: (i, 0)),
            pl.BlockSpec((_TCR, F_OUT), lambda i: (i, 0)),
        ],
        out_shape=[
            jax.ShapeDtypeStruct((N_PAD, F_OUT), jnp.float32),
            jax.ShapeDtypeStruct((N_PAD, F_OUT), jnp.float32),
        ],
    )(u1, sqd, dis, W2, b2)


def _tc3_body(u2_ref, sqd_ref, out_ref):
    out_ref[...] = u2_ref[...] * sqd_ref[...]


def _tc3(u2, sqd):
    grid = (N_PAD // _TCR,)
    return pl.pallas_call(
        _tc3_body,
        grid=grid,
        in_specs=[
            pl.BlockSpec((_TCR, F_OUT), lambda i: (i, 0)),
            pl.BlockSpec((_TCR, 1), lambda i: (i, 0)),
        ],
        out_specs=pl.BlockSpec((_TCR, F_OUT), lambda i: (i, 0)),
        out_shape=jax.ShapeDtypeStruct((N_PAD, F_OUT), jnp.float32),
    )(u2, sqd)


def _split_cols(a, Fc):
    return a.reshape(N_PAD, NC, Fc).transpose(1, 0, 2).reshape(NC * N_PAD, Fc)


def _merge_cols(a, Fc):
    return a.reshape(NC, N_PAD, Fc).transpose(1, 0, 2).reshape(N_PAD, NC * Fc)


@jax.jit
def kernel(x, edge_index, W1, b1, W2, b2):
    ei = edge_index.astype(jnp.int32)
    loop = jnp.arange(N, dtype=jnp.int32)
    src = jnp.concatenate([ei[0], loop])
    dst = jnp.concatenate([ei[1], loop])
    pad = E_PAD - src.shape[0]
    src = jnp.concatenate([src, jnp.full((pad,), DUMMY, jnp.int32)])
    dst = jnp.concatenate([dst, jnp.full((pad,), DUMMY, jnp.int32)])
    src_p = src.reshape(NS, EBg, EBG)
    dst_p = dst.reshape(NS, EBg, EBG)
    dst_w = dst.reshape(NC * NS, EBW, EBLK)
    x_pad = jnp.pad(x, ((0, N_PAD - N), (0, 0)))

    deg2 = _deg_call(dst_w)
    u0, u0c, w2c, sqd, dis = _tc1(deg2.T, x_pad, W1, b1.reshape(1, HIDDEN))
    w2v = w2c.reshape(N_PAD)

    u1 = _prop32(_split_cols(u0, HIDDEN // NC), _split_cols(u0c, HIDDEN // NC),
                 w2v, src_p, dst_p)
    u02, u0c2 = _tc2(_merge_cols(u1, HIDDEN // NC), sqd, dis,
                     W2, b2.reshape(1, F_OUT))
    u2 = _prop64(_split_cols(u02, F_OUT // NC), _split_cols(u0c2, F_OUT // NC),
                 w2v, src_p, dst_p)
    out = _tc3(_merge_cols(u2, F_OUT // NC), sqd)
    return out[:N]
